# baseline (speedup 1.0000x reference)
import functools

import jax
import jax.numpy as jnp
from jax import lax
from jax.experimental import pallas as pl
from jax.experimental.pallas import tpu as pltpu
from jax.experimental.pallas import tpu_sc as plsc

N = 10000
E = 320000
IN_DIM = 128
HID = 128
HEADS = 8
OUT = 64

NC = 2
NS = 16
NPAD = 10240
EP = 323584
BLK = 1024
PKSH = 14

_CH1 = EP // NS // 128
_CH2 = EP // (NC * NS) // 128
_RPT = NPAD // NS


def _tc1_body(h_ref, w_ref, uv_ref, z_ref, s_ref):
    hb = h_ref[...]
    for i in range(HEADS):
        z_ref[i] = jnp.dot(hb, w_ref[i], preferred_element_type=jnp.float32)
    s_ref[...] = jnp.dot(hb, uv_ref[...], preferred_element_type=jnp.float32)


def _tc1(hp, W1, UV):
    return pl.pallas_call(
        _tc1_body,
        grid=(NPAD // BLK,),
        in_specs=[
            pl.BlockSpec((BLK, IN_DIM), lambda j: (j, 0)),
            pl.BlockSpec((HEADS, IN_DIM, HID), lambda j: (0, 0, 0)),
            pl.BlockSpec((IN_DIM, 2 * HEADS), lambda j: (0, 0)),
        ],
        out_specs=[
            pl.BlockSpec((HEADS, BLK, HID), lambda j: (0, j, 0)),
            pl.BlockSpec((BLK, 2 * HEADS), lambda j: (j, 0)),
        ],
        out_shape=[
            jax.ShapeDtypeStruct((HEADS, NPAD, HID), jnp.float32),
            jax.ShapeDtypeStruct((NPAD, 2 * HEADS), jnp.float32),
        ],
    )(hp, W1, UV)


def _tc2_body(raw_ref, den_ref, w2_ref, a2_ref, z2_ref, s2_ref):
    acc = jnp.zeros((BLK, OUT), jnp.float32)
    for i in range(HEADS):
        d = jnp.sum(den_ref[i], axis=1, keepdims=True)
        x = raw_ref[i] * (1.0 / (d + 1e-9))
        x = jnp.where(x > 0, x, jnp.exp(x) - 1.0)
        acc = acc + jnp.dot(x, w2_ref[i], preferred_element_type=jnp.float32)
    z2_ref[...] = acc
    s2_ref[...] = jnp.dot(acc, a2_ref[...], preferred_element_type=jnp.float32)


def _tc2(out_raw, den1t, W2r, A2p):
    return pl.pallas_call(
        _tc2_body,
        grid=(NPAD // BLK,),
        in_specs=[
            pl.BlockSpec((HEADS, BLK, HID), lambda j: (0, j, 0)),
            pl.BlockSpec((HEADS, BLK, NS), lambda j: (0, j, 0)),
            pl.BlockSpec((HEADS, HID, OUT), lambda j: (0, 0, 0)),
            pl.BlockSpec((OUT, 8), lambda j: (0, 0)),
        ],
        out_specs=[
            pl.BlockSpec((BLK, OUT), lambda j: (j, 0)),
            pl.BlockSpec((BLK, 8), lambda j: (j, 0)),
        ],
        out_shape=[
            jax.ShapeDtypeStruct((NPAD, OUT), jnp.float32),
            jax.ShapeDtypeStruct((NPAD, 8), jnp.float32),
        ],
    )(out_raw, den1t, W2r, A2p)


def _tc3_body(p_ref, d_ref, o_ref):
    den = jnp.sum(d_ref[...], axis=1, keepdims=True)
    o_ref[...] = (p_ref[0, :, :OUT] + p_ref[1, :, :OUT]) * (1.0 / (den + 1e-9))


def _tc3(out2p, den2t):
    return pl.pallas_call(
        _tc3_body,
        grid=(NPAD // BLK,),
        in_specs=[
            pl.BlockSpec((2, BLK, HID), lambda j: (0, j, 0)),
            pl.BlockSpec((BLK, NC * NS), lambda j: (j, 0)),
        ],
        out_specs=pl.BlockSpec((BLK, OUT), lambda j: (j, 0)),
        out_shape=jax.ShapeDtypeStruct((NPAD, OUT), jnp.float32),
    )(out2p, den2t)


def _zero_vmem_2d(ref, nrows, ncols):
    zero16 = jnp.zeros((16,), jnp.float32)

    def body(r, carry):
        for g in range(ncols // 16):
            ref[r, pl.ds(g * 16, 16)] = zero16
        return carry

    lax.fori_loop(0, nrows, body, None)


def _zero_vmem_1d(ref, n):
    zero16 = jnp.zeros((16,), jnp.float32)

    def body(q, carry):
        ref[pl.ds(q * 16, 16)] = zero16
        return carry

    lax.fori_loop(0, n // 16, body, None)


def _unpack(pkv):
    srcv = lax.shift_right_logical(pkv, PKSH)
    dstv = lax.bitwise_and(pkv, (1 << PKSH) - 1)
    return srcv, dstv


def _phase_a(pk_c, w_c, sA, sB, den_part, nchunks):
    def body(j, carry):
        for g in range(8):
            sl = pl.ds(g * 16, 16)
            srcv, dstv = _unpack(pk_c[j, sl])
            x = plsc.load_gather(sA, [srcv]) + plsc.load_gather(sB, [dstv])
            wv = jnp.exp(jnp.maximum(x, 0.2 * x))
            w_c[j, sl] = wv
            plsc.addupdate_scatter(den_part, [dstv], wv)
        return carry

    lax.fori_loop(0, nchunks, body, None)


def _phase_b_pipe(pkE_tile, w_hbm_tile, bufs, acc_sh, zflat, nchunks,
                  ncols, idx_base):
    (pkA, pkB, idxA, idxB, dstA, dstB, wchA, wchB, rowsA, rowsB,
     psA, psB, wsA, wsB, gsA, gsB, ssA, ssB) = bufs

    def fire_pw(j, pk_ch, w_ch, ps, ws):
        pltpu.async_copy(pkE_tile.at[j], pk_ch, ps)
        pltpu.async_copy(w_hbm_tile.at[j], w_ch, ws)

    def wait_dma(src, dst, sem):
        pltpu.make_async_copy(src, dst, sem).wait()

    def prep(pk_ch, idx, dstr):
        for g in range(8):
            sl = pl.ds(g * 16, 16)
            srcv, dstv = _unpack(pk_ch[sl])
            idx[sl] = srcv + idx_base
            dstr[0, sl] = dstv

    def scale(rows, w_ch):
        def body(q, carry):
            for r in range(4):
                e = 4 * q + r
                wv = plsc.load_gather(w_ch, [jnp.full((16,), e, jnp.int32)])
                for g in range(ncols // 16):
                    sl = pl.ds(g * 16, 16)
                    rows[e, sl] = rows[e, sl] * wv
            return carry

        lax.fori_loop(0, 32, body, None)

    def scatter(rows, dstr):
        pltpu.sync_copy(rows, acc_sh.at[dstr.at[0]], add=True)

    def fire_scatter(rows, dstr, sem):
        pltpu.async_copy(rows, acc_sh.at[dstr.at[0]], sem, add=True)

    def wait_scatter(rows, dstr, sem):
        pltpu.make_async_copy(rows, acc_sh.at[dstr.at[0]], sem).wait()

    fire_pw(0, pkA, wchA, psA, wsA)
    fire_pw(1, pkB, wchB, psB, wsB)
    wait_dma(pkE_tile.at[0], pkA, psA)
    prep(pkA, idxA, dstA)
    pltpu.async_copy(zflat.at[idxA], rowsA, gsA)

    def body(p, carry):
        jB = 2 * p + 1

        @pl.when(p > 0)
        def _():
            wait_scatter(rowsB, dstB, ssB)

        wait_dma(pkE_tile.at[jB], pkB, psB)
        prep(pkB, idxB, dstB)
        pltpu.async_copy(zflat.at[idxB], rowsB, gsB)
        nxtA = 2 * p + 2

        wait_dma(zflat.at[idxA], rowsA, gsA)
        wait_dma(w_hbm_tile.at[0], wchA, wsA)
        scale(rowsA, wchA)

        @pl.when(nxtA < nchunks)
        def _():
            fire_pw(nxtA, pkA, wchA, psA, wsA)

        fire_scatter(rowsA, dstA, ssA)

        wait_dma(zflat.at[idxB], rowsB, gsB)
        wait_dma(w_hbm_tile.at[0], wchB, wsB)
        scale(rowsB, wchB)

        @pl.when(2 * p + 3 < nchunks)
        def _():
            fire_pw(2 * p + 3, pkB, wchB, psB, wsB)

        fire_scatter(rowsB, dstB, ssB)

        @pl.when(nxtA < nchunks)
        def _():
            wait_scatter(rowsA, dstA, ssA)
            wait_dma(pkE_tile.at[nxtA], pkA, psA)
            prep(pkA, idxA, dstA)
            pltpu.async_copy(zflat.at[idxA], rowsA, gsA)

        return carry

    lax.fori_loop(0, nchunks // 2, body, None)
    wait_scatter(rowsB, dstB, ssB)
    if nchunks % 2:
        wait_dma(zflat.at[idxA], rowsA, gsA)
        wait_dma(w_hbm_tile.at[0], wchA, wsA)
        scale(rowsA, wchA)
        scatter(rowsA, dstA)
    else:
        wait_scatter(rowsA, dstA, ssA)


def _acc_readout(acc_sh, rows, out_hbm_slice_fn, rbase):
    for b in range(_RPT // 128):
        pltpu.sync_copy(acc_sh.at[pl.ds(rbase + b * 128, 128)], rows)
        pltpu.sync_copy(rows, out_hbm_slice_fn(rbase + b * 128))


def _sca1_def(mesh):
    return functools.partial(
        pl.kernel,
        out_type=(
            jax.ShapeDtypeStruct((HEADS, NS, _CH1, 128), jnp.float32),
            jax.ShapeDtypeStruct((HEADS, NS, NPAD), jnp.float32),
        ),
        mesh=mesh,
        compiler_params=pltpu.CompilerParams(needs_layout_passes=False),
        scratch_types=[
            pltpu.VMEM((_CH1, 128), jnp.int32),
            pltpu.VMEM((_CH1, 128), jnp.float32),
            pltpu.VMEM((NPAD,), jnp.float32),
            pltpu.VMEM((NPAD,), jnp.float32),
            pltpu.VMEM((NPAD,), jnp.float32),
        ],
    )


def _sca1_body(pkE, ssT, sdT, wAll, denP, pk_c, w_c, sA, sB, den_part):
    c = lax.axis_index("c")
    s = lax.axis_index("s")
    pltpu.sync_copy(pkE.at[s], pk_c)
    for i in range(HEADS // NC):
        h_eff = c * (HEADS // NC) + i
        pltpu.sync_copy(ssT.at[h_eff], sA)
        pltpu.sync_copy(sdT.at[h_eff], sB)
        _zero_vmem_1d(den_part, NPAD)
        _phase_a(pk_c, w_c, sA, sB, den_part, _CH1)
        pltpu.sync_copy(w_c, wAll.at[h_eff, s])
        pltpu.sync_copy(den_part, denP.at[h_eff, s])


_SCB_SCRATCH = [
    pltpu.VMEM((128,), jnp.int32),
    pltpu.VMEM((128,), jnp.int32),
    pltpu.VMEM((128,), jnp.int32),
    pltpu.VMEM((128,), jnp.int32),
    pltpu.VMEM((1, 128), jnp.int32),
    pltpu.VMEM((1, 128), jnp.int32),
    pltpu.VMEM((128,), jnp.float32),
    pltpu.VMEM((128,), jnp.float32),
    pltpu.VMEM((128, HID), jnp.float32),
    pltpu.VMEM((128, HID), jnp.float32),
    pltpu.SemaphoreType.DMA,
    pltpu.SemaphoreType.DMA,
    pltpu.SemaphoreType.DMA,
    pltpu.SemaphoreType.DMA,
    pltpu.SemaphoreType.DMA,
    pltpu.SemaphoreType.DMA,
    pltpu.SemaphoreType.DMA,
    pltpu.SemaphoreType.DMA,
]


def _scb1_def(mesh):
    return functools.partial(
        pl.kernel,
        out_type=jax.ShapeDtypeStruct((HEADS, NPAD, HID), jnp.float32),
        mesh=mesh,
        compiler_params=pltpu.CompilerParams(needs_layout_passes=False),
        scratch_types=_SCB_SCRATCH
        + [pltpu.VMEM_SHARED((NPAD, HID), jnp.float32)],
    )


def _scb1_body(pkE, wAll, zflat, outR, *rest):
    bufs, acc_sh = rest[:-1], rest[-1]
    rowsA = bufs[8]
    c = lax.axis_index("c")
    s = lax.axis_index("s")
    rbase = s * _RPT
    for i in range(HEADS // NC):
        h_eff = c * (HEADS // NC) + i
        _zero_vmem_2d(rowsA, 128, HID)
        for b in range(_RPT // 128):
            pltpu.sync_copy(rowsA, acc_sh.at[pl.ds(rbase + b * 128, 128)])
        plsc.subcore_barrier()
        _phase_b_pipe(pkE.at[s], wAll.at[h_eff, s], bufs, acc_sh, zflat,
                      _CH1, HID, h_eff * NPAD)
        plsc.subcore_barrier()
        _acc_readout(acc_sh, rowsA,
                     lambda r: outR.at[h_eff, pl.ds(r, 128)], rbase)
        plsc.subcore_barrier()


def _sca2_def(mesh):
    return functools.partial(
        pl.kernel,
        out_type=(
            jax.ShapeDtypeStruct((NC * NS, _CH2, 128), jnp.float32),
            jax.ShapeDtypeStruct((NC * NS, NPAD), jnp.float32),
        ),
        mesh=mesh,
        compiler_params=pltpu.CompilerParams(needs_layout_passes=False),
        scratch_types=[
            pltpu.VMEM((_CH2, 128), jnp.int32),
            pltpu.VMEM((_CH2, 128), jnp.float32),
            pltpu.VMEM((NPAD,), jnp.float32),
            pltpu.VMEM((NPAD,), jnp.float32),
            pltpu.VMEM((NPAD,), jnp.float32),
        ],
    )


def _sca2_body(pkE, sS, sD, wAll, denP, pk_c, w_c, sA, sB, den_part):
    c = lax.axis_index("c")
    s = lax.axis_index("s")
    wid = c * NS + s
    pltpu.sync_copy(pkE.at[wid], pk_c)
    pltpu.sync_copy(sS, sA)
    pltpu.sync_copy(sD, sB)
    _zero_vmem_1d(den_part, NPAD)
    _phase_a(pk_c, w_c, sA, sB, den_part, _CH2)
    pltpu.sync_copy(w_c, wAll.at[wid])
    pltpu.sync_copy(den_part, denP.at[wid])


def _scb2_def(mesh):
    return functools.partial(
        pl.kernel,
        out_type=jax.ShapeDtypeStruct((NC, NPAD, HID), jnp.float32),
        mesh=mesh,
        compiler_params=pltpu.CompilerParams(needs_layout_passes=False),
        scratch_types=_SCB_SCRATCH
        + [pltpu.VMEM_SHARED((NPAD, HID), jnp.float32)],
    )


def _scb2_body(pkE, wAll, z2, outP, *rest):
    bufs, acc_sh = rest[:-1], rest[-1]
    rowsA = bufs[8]
    c = lax.axis_index("c")
    s = lax.axis_index("s")
    wid = c * NS + s
    rbase = s * _RPT
    _zero_vmem_2d(rowsA, 128, HID)
    for b in range(_RPT // 128):
        pltpu.sync_copy(rowsA, acc_sh.at[pl.ds(rbase + b * 128, 128)])
    plsc.subcore_barrier()
    _phase_b_pipe(pkE.at[wid], wAll.at[wid], bufs, acc_sh, z2,
                  _CH2, HID, 0)
    plsc.subcore_barrier()
    _acc_readout(acc_sh, rowsA, lambda r: outP.at[c, pl.ds(r, 128)], rbase)


_SC_CACHE = {}


def _sc_kernels():
    if "k" not in _SC_CACHE:
        mesh = plsc.VectorSubcoreMesh(core_axis_name="c", subcore_axis_name="s",
                                      num_cores=NC, num_subcores=NS)
        _SC_CACHE["k"] = (
            _sca1_def(mesh)(_sca1_body),
            _scb1_def(mesh)(_scb1_body),
            _sca2_def(mesh)(_sca2_body),
            _scb2_def(mesh)(_scb2_body),
        )
    return _SC_CACHE["k"]


def kernel(h, edge_index, W1, a1, W2, a2):
    _sca1, _scb1, _sca2, _scb2 = _sc_kernels()
    h = h.astype(jnp.float32)
    src = edge_index[0].astype(jnp.int32)
    dst = edge_index[1].astype(jnp.int32)
    npad_e = EP - E
    srcp = jnp.concatenate([src, jnp.zeros((npad_e,), jnp.int32)])
    dstp = jnp.concatenate([dst, jnp.full((npad_e,), NPAD - 1, jnp.int32)])
    pk = srcp * (1 << PKSH) + dstp
    pk1 = pk.reshape(NS, _CH1, 128)
    pk2 = pk.reshape(NC * NS, _CH2, 128)

    hp = jnp.pad(h, ((0, NPAD - N), (0, 0)))
    U = jnp.einsum("hio,ho->ih", W1, a1[:, :HID])
    V = jnp.einsum("hio,ho->ih", W1, a1[:, HID:])
    UV = jnp.concatenate([U, V], axis=1)

    z1, S1 = _tc1(hp, W1, UV)
    ssT = S1[:, :HEADS].T
    sdT = S1[:, HEADS:].T
    z1flat = z1.reshape(HEADS * NPAD, HID)

    w1, den1p = _sca1(pk1, ssT, sdT)
    out_raw = _scb1(pk1, w1, z1flat)

    W2r = W2.reshape(HEADS, HID, OUT)
    A2p = jnp.pad(jnp.stack([a2[0, :OUT], a2[0, OUT:]], axis=1),
                  ((0, 0), (0, 6)))
    den1t = jnp.transpose(den1p, (0, 2, 1))
    z2, S2 = _tc2(out_raw, den1t, W2r, A2p)

    w2, den2p = _sca2(pk2, S2[:, 0], S2[:, 1])
    z2p = jnp.pad(z2, ((0, 0), (0, HID - OUT)))
    out2p = _scb2(pk2, w2, z2p)
    out = _tc3(out2p, den2p.T)
    return out[:N]


# speedup vs baseline: 1.0473x; 1.0473x over previous
import functools

import jax
import jax.numpy as jnp
from jax import lax
from jax.experimental import pallas as pl
from jax.experimental.pallas import tpu as pltpu
from jax.experimental.pallas import tpu_sc as plsc

N = 10000
E = 320000
IN_DIM = 128
HID = 128
HEADS = 8
OUT = 64

NC = 2
NS = 16
NPAD = 10240
EP = 323584
BLK = 1024
PKSH = 14

_CH1 = EP // NS // 128
_CH2 = EP // (NC * NS) // 128
_RPT = NPAD // NS


def _tc1_body(h_ref, w_ref, uv_ref, z_ref, s_ref):
    hb = h_ref[...]
    for i in range(HEADS):
        z_ref[i] = jnp.dot(hb, w_ref[i], preferred_element_type=jnp.float32)
    s_ref[...] = jnp.dot(hb, uv_ref[...], preferred_element_type=jnp.float32)


def _tc1(hp, W1, UV):
    return pl.pallas_call(
        _tc1_body,
        grid=(NPAD // BLK,),
        in_specs=[
            pl.BlockSpec((BLK, IN_DIM), lambda j: (j, 0)),
            pl.BlockSpec((HEADS, IN_DIM, HID), lambda j: (0, 0, 0)),
            pl.BlockSpec((IN_DIM, 2 * HEADS), lambda j: (0, 0)),
        ],
        out_specs=[
            pl.BlockSpec((HEADS, BLK, HID), lambda j: (0, j, 0)),
            pl.BlockSpec((BLK, 2 * HEADS), lambda j: (j, 0)),
        ],
        out_shape=[
            jax.ShapeDtypeStruct((HEADS, NPAD, HID), jnp.float32),
            jax.ShapeDtypeStruct((NPAD, 2 * HEADS), jnp.float32),
        ],
    )(hp, W1, UV)


def _tc2_body(raw_ref, den_ref, w2_ref, a2_ref, z2_ref, s2_ref):
    acc = jnp.zeros((BLK, OUT), jnp.float32)
    for i in range(HEADS):
        d = jnp.sum(den_ref[i], axis=1, keepdims=True)
        x = raw_ref[i] * (1.0 / (d + 1e-9))
        x = jnp.where(x > 0, x, jnp.exp(x) - 1.0)
        acc = acc + jnp.dot(x, w2_ref[i], preferred_element_type=jnp.float32)
    z2_ref[...] = acc
    s2_ref[...] = jnp.dot(acc, a2_ref[...], preferred_element_type=jnp.float32)


def _tc2(out_raw, den1t, W2r, A2p):
    return pl.pallas_call(
        _tc2_body,
        grid=(NPAD // BLK,),
        in_specs=[
            pl.BlockSpec((HEADS, BLK, HID), lambda j: (0, j, 0)),
            pl.BlockSpec((HEADS, BLK, NS), lambda j: (0, j, 0)),
            pl.BlockSpec((HEADS, HID, OUT), lambda j: (0, 0, 0)),
            pl.BlockSpec((OUT, 8), lambda j: (0, 0)),
        ],
        out_specs=[
            pl.BlockSpec((BLK, OUT), lambda j: (j, 0)),
            pl.BlockSpec((BLK, 8), lambda j: (j, 0)),
        ],
        out_shape=[
            jax.ShapeDtypeStruct((NPAD, OUT), jnp.float32),
            jax.ShapeDtypeStruct((NPAD, 8), jnp.float32),
        ],
    )(out_raw, den1t, W2r, A2p)


def _tc3_body(p_ref, d_ref, o_ref):
    den = jnp.sum(d_ref[...], axis=1, keepdims=True)
    o_ref[...] = (p_ref[0, :, :OUT] + p_ref[1, :, :OUT]) * (1.0 / (den + 1e-9))


def _tc3(out2p, den2t):
    return pl.pallas_call(
        _tc3_body,
        grid=(NPAD // BLK,),
        in_specs=[
            pl.BlockSpec((2, BLK, HID), lambda j: (0, j, 0)),
            pl.BlockSpec((BLK, NC * NS), lambda j: (j, 0)),
        ],
        out_specs=pl.BlockSpec((BLK, OUT), lambda j: (j, 0)),
        out_shape=jax.ShapeDtypeStruct((NPAD, OUT), jnp.float32),
    )(out2p, den2t)


def _zero_vmem_2d(ref, nrows, ncols):
    zero16 = jnp.zeros((16,), jnp.float32)

    def body(r, carry):
        for g in range(ncols // 16):
            ref[r, pl.ds(g * 16, 16)] = zero16
        return carry

    lax.fori_loop(0, nrows, body, None)


def _zero_vmem_1d(ref, n):
    zero16 = jnp.zeros((16,), jnp.float32)

    def body(q, carry):
        ref[pl.ds(q * 16, 16)] = zero16
        return carry

    lax.fori_loop(0, n // 16, body, None)


def _unpack(pkv):
    srcv = lax.shift_right_logical(pkv, PKSH)
    dstv = lax.bitwise_and(pkv, (1 << PKSH) - 1)
    return srcv, dstv


def _phase_a(pk_c, w_c, sA, sB, den_part, nchunks):
    def body(j, carry):
        for g in range(8):
            sl = pl.ds(g * 16, 16)
            srcv, dstv = _unpack(pk_c[j, sl])
            x = plsc.load_gather(sA, [srcv]) + plsc.load_gather(sB, [dstv])
            wv = jnp.exp(jnp.maximum(x, 0.2 * x))
            w_c[j, sl] = wv
            plsc.addupdate_scatter(den_part, [dstv], wv)
        return carry

    lax.fori_loop(0, nchunks, body, None)


def _phase_b_pipe(pkE_tile, w_hbm_tile, bufs, acc_sh, zflat, nchunks,
                  ncols, idx_base):
    (pkA, pkB, idxA, idxB, dstA, dstB, wchA, wchB, rowsA, rowsB,
     psA, psB, wsA, wsB, gsA, gsB, _ssA, _ssB) = bufs

    def fire_pw(j, pk_ch, w_ch, ps, ws):
        pltpu.async_copy(pkE_tile.at[j], pk_ch, ps)
        pltpu.async_copy(w_hbm_tile.at[j], w_ch, ws)

    def wait_dma(src, dst, sem):
        pltpu.make_async_copy(src, dst, sem).wait()

    def prep(pk_ch, idx, dstr):
        for g in range(8):
            sl = pl.ds(g * 16, 16)
            srcv, dstv = _unpack(pk_ch[sl])
            idx[sl] = srcv + idx_base
            dstr[0, sl] = dstv

    def scale(rows, w_ch):
        def body(q, carry):
            for r in range(4):
                e = 4 * q + r
                wv = plsc.load_gather(w_ch, [jnp.full((16,), e, jnp.int32)])
                for g in range(ncols // 16):
                    sl = pl.ds(g * 16, 16)
                    rows[e, sl] = rows[e, sl] * wv
            return carry

        lax.fori_loop(0, 32, body, None)

    def scatter(rows, dstr):
        pltpu.sync_copy(rows, acc_sh.at[dstr.at[0]], add=True)

    fire_pw(0, pkA, wchA, psA, wsA)
    fire_pw(1, pkB, wchB, psB, wsB)
    wait_dma(pkE_tile.at[0], pkA, psA)
    prep(pkA, idxA, dstA)
    pltpu.async_copy(zflat.at[idxA], rowsA, gsA)

    def body(p, carry):
        jB = 2 * p + 1
        wait_dma(pkE_tile.at[jB], pkB, psB)
        prep(pkB, idxB, dstB)
        pltpu.async_copy(zflat.at[idxB], rowsB, gsB)
        nxtA = 2 * p + 2

        wait_dma(zflat.at[idxA], rowsA, gsA)
        wait_dma(w_hbm_tile.at[0], wchA, wsA)
        scale(rowsA, wchA)

        @pl.when(nxtA < nchunks)
        def _():
            fire_pw(nxtA, pkA, wchA, psA, wsA)

        scatter(rowsA, dstA)

        @pl.when(nxtA < nchunks)
        def _():
            wait_dma(pkE_tile.at[nxtA], pkA, psA)
            prep(pkA, idxA, dstA)
            pltpu.async_copy(zflat.at[idxA], rowsA, gsA)

        wait_dma(zflat.at[idxB], rowsB, gsB)
        wait_dma(w_hbm_tile.at[0], wchB, wsB)
        scale(rowsB, wchB)

        @pl.when(2 * p + 3 < nchunks)
        def _():
            fire_pw(2 * p + 3, pkB, wchB, psB, wsB)

        scatter(rowsB, dstB)
        return carry

    lax.fori_loop(0, nchunks // 2, body, None)
    if nchunks % 2:
        wait_dma(zflat.at[idxA], rowsA, gsA)
        wait_dma(w_hbm_tile.at[0], wchA, wsA)
        scale(rowsA, wchA)
        scatter(rowsA, dstA)


def _acc_readout(acc_sh, rows, out_hbm_slice_fn, rbase):
    for b in range(_RPT // 128):
        pltpu.sync_copy(acc_sh.at[pl.ds(rbase + b * 128, 128)], rows)
        pltpu.sync_copy(rows, out_hbm_slice_fn(rbase + b * 128))


def _sca1_def(mesh):
    return functools.partial(
        pl.kernel,
        out_type=(
            jax.ShapeDtypeStruct((HEADS, NS, _CH1, 128), jnp.float32),
            jax.ShapeDtypeStruct((HEADS, NS, NPAD), jnp.float32),
        ),
        mesh=mesh,
        compiler_params=pltpu.CompilerParams(needs_layout_passes=False),
        scratch_types=[
            pltpu.VMEM((_CH1, 128), jnp.int32),
            pltpu.VMEM((_CH1, 128), jnp.float32),
            pltpu.VMEM((NPAD,), jnp.float32),
            pltpu.VMEM((NPAD,), jnp.float32),
            pltpu.VMEM((NPAD,), jnp.float32),
        ],
    )


def _sca1_body(pkE, ssT, sdT, wAll, denP, pk_c, w_c, sA, sB, den_part):
    c = lax.axis_index("c")
    s = lax.axis_index("s")
    pltpu.sync_copy(pkE.at[s], pk_c)
    for i in range(HEADS // NC):
        h_eff = c * (HEADS // NC) + i
        pltpu.sync_copy(ssT.at[h_eff], sA)
        pltpu.sync_copy(sdT.at[h_eff], sB)
        _zero_vmem_1d(den_part, NPAD)
        _phase_a(pk_c, w_c, sA, sB, den_part, _CH1)
        pltpu.sync_copy(w_c, wAll.at[h_eff, s])
        pltpu.sync_copy(den_part, denP.at[h_eff, s])


_SCB_SCRATCH = [
    pltpu.VMEM((128,), jnp.int32),
    pltpu.VMEM((128,), jnp.int32),
    pltpu.VMEM((128,), jnp.int32),
    pltpu.VMEM((128,), jnp.int32),
    pltpu.VMEM((1, 128), jnp.int32),
    pltpu.VMEM((1, 128), jnp.int32),
    pltpu.VMEM((128,), jnp.float32),
    pltpu.VMEM((128,), jnp.float32),
    pltpu.VMEM((128, HID), jnp.float32),
    pltpu.VMEM((128, HID), jnp.float32),
    pltpu.SemaphoreType.DMA,
    pltpu.SemaphoreType.DMA,
    pltpu.SemaphoreType.DMA,
    pltpu.SemaphoreType.DMA,
    pltpu.SemaphoreType.DMA,
    pltpu.SemaphoreType.DMA,
    pltpu.SemaphoreType.DMA,
    pltpu.SemaphoreType.DMA,
]


def _scb1_def(mesh):
    return functools.partial(
        pl.kernel,
        out_type=jax.ShapeDtypeStruct((HEADS, NPAD, HID), jnp.float32),
        mesh=mesh,
        compiler_params=pltpu.CompilerParams(needs_layout_passes=False),
        scratch_types=_SCB_SCRATCH
        + [pltpu.VMEM_SHARED((NPAD, HID), jnp.float32)],
    )


def _scb1_body(pkE, wAll, zflat, outR, *rest):
    bufs, acc_sh = rest[:-1], rest[-1]
    rowsA = bufs[8]
    c = lax.axis_index("c")
    s = lax.axis_index("s")
    rbase = s * _RPT
    for i in range(HEADS // NC):
        h_eff = c * (HEADS // NC) + i
        _zero_vmem_2d(rowsA, 128, HID)
        for b in range(_RPT // 128):
            pltpu.sync_copy(rowsA, acc_sh.at[pl.ds(rbase + b * 128, 128)])
        plsc.subcore_barrier()
        _phase_b_pipe(pkE.at[s], wAll.at[h_eff, s], bufs, acc_sh, zflat,
                      _CH1, HID, h_eff * NPAD)
        plsc.subcore_barrier()
        _acc_readout(acc_sh, rowsA,
                     lambda r: outR.at[h_eff, pl.ds(r, 128)], rbase)
        plsc.subcore_barrier()


def _sca2_def(mesh):
    return functools.partial(
        pl.kernel,
        out_type=(
            jax.ShapeDtypeStruct((NC * NS, _CH2, 128), jnp.float32),
            jax.ShapeDtypeStruct((NC * NS, NPAD), jnp.float32),
        ),
        mesh=mesh,
        compiler_params=pltpu.CompilerParams(needs_layout_passes=False),
        scratch_types=[
            pltpu.VMEM((_CH2, 128), jnp.int32),
            pltpu.VMEM((_CH2, 128), jnp.float32),
            pltpu.VMEM((NPAD,), jnp.float32),
            pltpu.VMEM((NPAD,), jnp.float32),
            pltpu.VMEM((NPAD,), jnp.float32),
        ],
    )


def _sca2_body(pkE, sS, sD, wAll, denP, pk_c, w_c, sA, sB, den_part):
    c = lax.axis_index("c")
    s = lax.axis_index("s")
    wid = c * NS + s
    pltpu.sync_copy(pkE.at[wid], pk_c)
    pltpu.sync_copy(sS, sA)
    pltpu.sync_copy(sD, sB)
    _zero_vmem_1d(den_part, NPAD)
    _phase_a(pk_c, w_c, sA, sB, den_part, _CH2)
    pltpu.sync_copy(w_c, wAll.at[wid])
    pltpu.sync_copy(den_part, denP.at[wid])


def _scb2_def(mesh):
    return functools.partial(
        pl.kernel,
        out_type=jax.ShapeDtypeStruct((NC, NPAD, HID), jnp.float32),
        mesh=mesh,
        compiler_params=pltpu.CompilerParams(needs_layout_passes=False),
        scratch_types=_SCB_SCRATCH
        + [pltpu.VMEM_SHARED((NPAD, HID), jnp.float32)],
    )


def _scb2_body(pkE, wAll, z2, outP, *rest):
    bufs, acc_sh = rest[:-1], rest[-1]
    rowsA = bufs[8]
    c = lax.axis_index("c")
    s = lax.axis_index("s")
    wid = c * NS + s
    rbase = s * _RPT
    _zero_vmem_2d(rowsA, 128, HID)
    for b in range(_RPT // 128):
        pltpu.sync_copy(rowsA, acc_sh.at[pl.ds(rbase + b * 128, 128)])
    plsc.subcore_barrier()
    _phase_b_pipe(pkE.at[wid], wAll.at[wid], bufs, acc_sh, z2,
                  _CH2, HID, 0)
    plsc.subcore_barrier()
    _acc_readout(acc_sh, rowsA, lambda r: outP.at[c, pl.ds(r, 128)], rbase)


_SC_CACHE = {}


def _sc_kernels():
    if "k" not in _SC_CACHE:
        mesh = plsc.VectorSubcoreMesh(core_axis_name="c", subcore_axis_name="s",
                                      num_cores=NC, num_subcores=NS)
        _SC_CACHE["k"] = (
            _sca1_def(mesh)(_sca1_body),
            _scb1_def(mesh)(_scb1_body),
            _sca2_def(mesh)(_sca2_body),
            _scb2_def(mesh)(_scb2_body),
        )
    return _SC_CACHE["k"]


def kernel(h, edge_index, W1, a1, W2, a2):
    _sca1, _scb1, _sca2, _scb2 = _sc_kernels()
    h = h.astype(jnp.float32)
    src = edge_index[0].astype(jnp.int32)
    dst = edge_index[1].astype(jnp.int32)
    npad_e = EP - E
    srcp = jnp.concatenate([src, jnp.zeros((npad_e,), jnp.int32)])
    dstp = jnp.concatenate([dst, jnp.full((npad_e,), NPAD - 1, jnp.int32)])
    pk = srcp * (1 << PKSH) + dstp
    pk1 = pk.reshape(NS, _CH1, 128)
    pk2 = pk.reshape(NC * NS, _CH2, 128)

    hp = jnp.pad(h, ((0, NPAD - N), (0, 0)))
    U = jnp.einsum("hio,ho->ih", W1, a1[:, :HID])
    V = jnp.einsum("hio,ho->ih", W1, a1[:, HID:])
    UV = jnp.concatenate([U, V], axis=1)

    z1, S1 = _tc1(hp, W1, UV)
    ssT = S1[:, :HEADS].T
    sdT = S1[:, HEADS:].T
    z1flat = z1.reshape(HEADS * NPAD, HID)

    w1, den1p = _sca1(pk1, ssT, sdT)
    out_raw = _scb1(pk1, w1, z1flat)

    W2r = W2.reshape(HEADS, HID, OUT)
    A2p = jnp.pad(jnp.stack([a2[0, :OUT], a2[0, OUT:]], axis=1),
                  ((0, 0), (0, 6)))
    den1t = jnp.transpose(den1p, (0, 2, 1))
    z2, S2 = _tc2(out_raw, den1t, W2r, A2p)

    w2, den2p = _sca2(pk2, S2[:, 0], S2[:, 1])
    z2p = jnp.pad(z2, ((0, 0), (0, HID - OUT)))
    out2p = _scb2(pk2, w2, z2p)
    out = _tc3(out2p, den2p.T)
    return out[:N]


# speedup vs baseline: 1.0481x; 1.0008x over previous
import functools

import jax
import jax.numpy as jnp
from jax import lax
from jax.experimental import pallas as pl
from jax.experimental.pallas import tpu as pltpu
from jax.experimental.pallas import tpu_sc as plsc

N = 10000
E = 320000
IN_DIM = 128
HID = 128
HEADS = 8
OUT = 64

NC = 2
NS = 16
NPAD = 10240
EP = 323584
BLK = 1024
PKSH = 14

_CH1 = EP // NS // 128
_CH2 = EP // (NC * NS) // 128
_RPT = NPAD // NS


def _tc1_body(h_ref, w_ref, uv_ref, z_ref, s_ref):
    hb = h_ref[...]
    for i in range(HEADS):
        z_ref[i] = jnp.dot(hb, w_ref[i], preferred_element_type=jnp.float32)
    s_ref[...] = jnp.dot(hb, uv_ref[...], preferred_element_type=jnp.float32)


def _tc1(hp, W1, UV):
    return pl.pallas_call(
        _tc1_body,
        grid=(NPAD // BLK,),
        in_specs=[
            pl.BlockSpec((BLK, IN_DIM), lambda j: (j, 0)),
            pl.BlockSpec((HEADS, IN_DIM, HID), lambda j: (0, 0, 0)),
            pl.BlockSpec((IN_DIM, 2 * HEADS), lambda j: (0, 0)),
        ],
        out_specs=[
            pl.BlockSpec((HEADS, BLK, HID), lambda j: (0, j, 0)),
            pl.BlockSpec((BLK, 2 * HEADS), lambda j: (j, 0)),
        ],
        out_shape=[
            jax.ShapeDtypeStruct((HEADS, NPAD, HID), jnp.float32),
            jax.ShapeDtypeStruct((NPAD, 2 * HEADS), jnp.float32),
        ],
    )(hp, W1, UV)


def _tc2_body(raw_ref, den_ref, w2_ref, a2_ref, z2_ref, s2_ref):
    acc = jnp.zeros((BLK, OUT), jnp.float32)
    for i in range(HEADS):
        d = jnp.sum(den_ref[i], axis=1, keepdims=True)
        x = raw_ref[i] * (1.0 / (d + 1e-9))
        x = jnp.where(x > 0, x, jnp.exp(x) - 1.0)
        acc = acc + jnp.dot(x, w2_ref[i], preferred_element_type=jnp.float32)
    z2_ref[...] = acc
    s2_ref[...] = jnp.dot(acc, a2_ref[...], preferred_element_type=jnp.float32)


def _tc2(out_raw, den1t, W2r, A2p):
    return pl.pallas_call(
        _tc2_body,
        grid=(NPAD // BLK,),
        in_specs=[
            pl.BlockSpec((HEADS, BLK, HID), lambda j: (0, j, 0)),
            pl.BlockSpec((HEADS, BLK, NS), lambda j: (0, j, 0)),
            pl.BlockSpec((HEADS, HID, OUT), lambda j: (0, 0, 0)),
            pl.BlockSpec((OUT, 8), lambda j: (0, 0)),
        ],
        out_specs=[
            pl.BlockSpec((BLK, OUT), lambda j: (j, 0)),
            pl.BlockSpec((BLK, 8), lambda j: (j, 0)),
        ],
        out_shape=[
            jax.ShapeDtypeStruct((NPAD, OUT), jnp.float32),
            jax.ShapeDtypeStruct((NPAD, 8), jnp.float32),
        ],
    )(out_raw, den1t, W2r, A2p)


def _tc3_body(p_ref, d_ref, o_ref):
    den = jnp.sum(d_ref[...], axis=1, keepdims=True)
    o_ref[...] = (p_ref[0, :, :OUT] + p_ref[1, :, :OUT]) * (1.0 / (den + 1e-9))


def _tc3(out2p, den2t):
    return pl.pallas_call(
        _tc3_body,
        grid=(NPAD // BLK,),
        in_specs=[
            pl.BlockSpec((2, BLK, HID), lambda j: (0, j, 0)),
            pl.BlockSpec((BLK, NC * NS), lambda j: (j, 0)),
        ],
        out_specs=pl.BlockSpec((BLK, OUT), lambda j: (j, 0)),
        out_shape=jax.ShapeDtypeStruct((NPAD, OUT), jnp.float32),
    )(out2p, den2t)


def _zero_vmem_2d(ref, nrows, ncols):
    zero16 = jnp.zeros((16,), jnp.float32)

    def body(r, carry):
        for g in range(ncols // 16):
            ref[r, pl.ds(g * 16, 16)] = zero16
        return carry

    lax.fori_loop(0, nrows, body, None)


def _zero_vmem_1d(ref, n):
    zero16 = jnp.zeros((16,), jnp.float32)

    def body(q, carry):
        ref[pl.ds(q * 16, 16)] = zero16
        return carry

    lax.fori_loop(0, n // 16, body, None)


def _unpack(pkv):
    srcv = lax.shift_right_logical(pkv, PKSH)
    dstv = lax.bitwise_and(pkv, (1 << PKSH) - 1)
    return srcv, dstv


def _phase_a(pk_c, w_c, sA, sB, den_part, nchunks):
    def body(j, carry):
        for g in range(8):
            sl = pl.ds(g * 16, 16)
            srcv, dstv = _unpack(pk_c[j, sl])
            x = plsc.load_gather(sA, [srcv]) + plsc.load_gather(sB, [dstv])
            wv = jnp.exp(jnp.maximum(x, 0.2 * x))
            w_c[j, sl] = wv
            plsc.addupdate_scatter(den_part, [dstv], wv)
        return carry

    lax.fori_loop(0, nchunks, body, None)


def _phase_b_pipe(pkE_tile, w_hbm_tile, bufs, acc_sh, zflat, nchunks,
                  ncols, idx_base):
    (pkA, pkB, idxA, idxB, dstA, dstB, wchA, wchB, rowsA, rowsB,
     psA, psB, wsA, wsB, gsA, gsB, _ssA, _ssB) = bufs

    def fire_pw(j, pk_ch, w_ch, ps, ws):
        pltpu.async_copy(pkE_tile.at[j], pk_ch, ps)
        pltpu.async_copy(w_hbm_tile.at[j], w_ch, ws)

    def wait_dma(src, dst, sem):
        pltpu.make_async_copy(src, dst, sem).wait()

    def prep(pk_ch, idx, dstr):
        for g in range(8):
            sl = pl.ds(g * 16, 16)
            srcv, dstv = _unpack(pk_ch[sl])
            idx[sl] = srcv + idx_base
            dstr[0, sl] = dstv

    def scale(rows, w_ch):
        def body(q, carry):
            for r in range(8):
                e = 8 * q + r
                wv = plsc.load_gather(w_ch, [jnp.full((16,), e, jnp.int32)])
                for g in range(ncols // 16):
                    sl = pl.ds(g * 16, 16)
                    rows[e, sl] = rows[e, sl] * wv
            return carry

        lax.fori_loop(0, 16, body, None)

    def scatter(rows, dstr):
        pltpu.sync_copy(rows, acc_sh.at[dstr.at[0]], add=True)

    fire_pw(0, pkA, wchA, psA, wsA)
    fire_pw(1, pkB, wchB, psB, wsB)
    wait_dma(pkE_tile.at[0], pkA, psA)
    prep(pkA, idxA, dstA)
    pltpu.async_copy(zflat.at[idxA], rowsA, gsA)

    def body(p, carry):
        jB = 2 * p + 1
        wait_dma(pkE_tile.at[jB], pkB, psB)
        prep(pkB, idxB, dstB)
        pltpu.async_copy(zflat.at[idxB], rowsB, gsB)
        nxtA = 2 * p + 2

        wait_dma(zflat.at[idxA], rowsA, gsA)
        wait_dma(w_hbm_tile.at[0], wchA, wsA)
        scale(rowsA, wchA)

        @pl.when(nxtA < nchunks)
        def _():
            fire_pw(nxtA, pkA, wchA, psA, wsA)

        scatter(rowsA, dstA)

        @pl.when(nxtA < nchunks)
        def _():
            wait_dma(pkE_tile.at[nxtA], pkA, psA)
            prep(pkA, idxA, dstA)
            pltpu.async_copy(zflat.at[idxA], rowsA, gsA)

        wait_dma(zflat.at[idxB], rowsB, gsB)
        wait_dma(w_hbm_tile.at[0], wchB, wsB)
        scale(rowsB, wchB)

        @pl.when(2 * p + 3 < nchunks)
        def _():
            fire_pw(2 * p + 3, pkB, wchB, psB, wsB)

        scatter(rowsB, dstB)
        return carry

    lax.fori_loop(0, nchunks // 2, body, None)
    if nchunks % 2:
        wait_dma(zflat.at[idxA], rowsA, gsA)
        wait_dma(w_hbm_tile.at[0], wchA, wsA)
        scale(rowsA, wchA)
        scatter(rowsA, dstA)


def _acc_readout(acc_sh, rows, out_hbm_slice_fn, rbase):
    for b in range(_RPT // 128):
        pltpu.sync_copy(acc_sh.at[pl.ds(rbase + b * 128, 128)], rows)
        pltpu.sync_copy(rows, out_hbm_slice_fn(rbase + b * 128))


def _sca1_def(mesh):
    return functools.partial(
        pl.kernel,
        out_type=(
            jax.ShapeDtypeStruct((HEADS, NS, _CH1, 128), jnp.float32),
            jax.ShapeDtypeStruct((HEADS, NS, NPAD), jnp.float32),
        ),
        mesh=mesh,
        compiler_params=pltpu.CompilerParams(needs_layout_passes=False),
        scratch_types=[
            pltpu.VMEM((_CH1, 128), jnp.int32),
            pltpu.VMEM((_CH1, 128), jnp.float32),
            pltpu.VMEM((NPAD,), jnp.float32),
            pltpu.VMEM((NPAD,), jnp.float32),
            pltpu.VMEM((NPAD,), jnp.float32),
        ],
    )


def _sca1_body(pkE, ssT, sdT, wAll, denP, pk_c, w_c, sA, sB, den_part):
    c = lax.axis_index("c")
    s = lax.axis_index("s")
    pltpu.sync_copy(pkE.at[s], pk_c)
    for i in range(HEADS // NC):
        h_eff = c * (HEADS // NC) + i
        pltpu.sync_copy(ssT.at[h_eff], sA)
        pltpu.sync_copy(sdT.at[h_eff], sB)
        _zero_vmem_1d(den_part, NPAD)
        _phase_a(pk_c, w_c, sA, sB, den_part, _CH1)
        pltpu.sync_copy(w_c, wAll.at[h_eff, s])
        pltpu.sync_copy(den_part, denP.at[h_eff, s])


_SCB_SCRATCH = [
    pltpu.VMEM((128,), jnp.int32),
    pltpu.VMEM((128,), jnp.int32),
    pltpu.VMEM((128,), jnp.int32),
    pltpu.VMEM((128,), jnp.int32),
    pltpu.VMEM((1, 128), jnp.int32),
    pltpu.VMEM((1, 128), jnp.int32),
    pltpu.VMEM((128,), jnp.float32),
    pltpu.VMEM((128,), jnp.float32),
    pltpu.VMEM((128, HID), jnp.float32),
    pltpu.VMEM((128, HID), jnp.float32),
    pltpu.SemaphoreType.DMA,
    pltpu.SemaphoreType.DMA,
    pltpu.SemaphoreType.DMA,
    pltpu.SemaphoreType.DMA,
    pltpu.SemaphoreType.DMA,
    pltpu.SemaphoreType.DMA,
    pltpu.SemaphoreType.DMA,
    pltpu.SemaphoreType.DMA,
]


def _scb1_def(mesh):
    return functools.partial(
        pl.kernel,
        out_type=jax.ShapeDtypeStruct((HEADS, NPAD, HID), jnp.float32),
        mesh=mesh,
        compiler_params=pltpu.CompilerParams(needs_layout_passes=False),
        scratch_types=_SCB_SCRATCH
        + [pltpu.VMEM_SHARED((NPAD, HID), jnp.float32)],
    )


def _scb1_body(pkE, wAll, zflat, outR, *rest):
    bufs, acc_sh = rest[:-1], rest[-1]
    rowsA = bufs[8]
    c = lax.axis_index("c")
    s = lax.axis_index("s")
    rbase = s * _RPT
    for i in range(HEADS // NC):
        h_eff = c * (HEADS // NC) + i
        _zero_vmem_2d(rowsA, 128, HID)
        for b in range(_RPT // 128):
            pltpu.sync_copy(rowsA, acc_sh.at[pl.ds(rbase + b * 128, 128)])
        plsc.subcore_barrier()
        _phase_b_pipe(pkE.at[s], wAll.at[h_eff, s], bufs, acc_sh, zflat,
                      _CH1, HID, h_eff * NPAD)
        plsc.subcore_barrier()
        _acc_readout(acc_sh, rowsA,
                     lambda r: outR.at[h_eff, pl.ds(r, 128)], rbase)
        plsc.subcore_barrier()


def _sca2_def(mesh):
    return functools.partial(
        pl.kernel,
        out_type=(
            jax.ShapeDtypeStruct((NC * NS, _CH2, 128), jnp.float32),
            jax.ShapeDtypeStruct((NC * NS, NPAD), jnp.float32),
        ),
        mesh=mesh,
        compiler_params=pltpu.CompilerParams(needs_layout_passes=False),
        scratch_types=[
            pltpu.VMEM((_CH2, 128), jnp.int32),
            pltpu.VMEM((_CH2, 128), jnp.float32),
            pltpu.VMEM((NPAD,), jnp.float32),
            pltpu.VMEM((NPAD,), jnp.float32),
            pltpu.VMEM((NPAD,), jnp.float32),
        ],
    )


def _sca2_body(pkE, sS, sD, wAll, denP, pk_c, w_c, sA, sB, den_part):
    c = lax.axis_index("c")
    s = lax.axis_index("s")
    wid = c * NS + s
    pltpu.sync_copy(pkE.at[wid], pk_c)
    pltpu.sync_copy(sS, sA)
    pltpu.sync_copy(sD, sB)
    _zero_vmem_1d(den_part, NPAD)
    _phase_a(pk_c, w_c, sA, sB, den_part, _CH2)
    pltpu.sync_copy(w_c, wAll.at[wid])
    pltpu.sync_copy(den_part, denP.at[wid])


def _scb2_def(mesh):
    return functools.partial(
        pl.kernel,
        out_type=jax.ShapeDtypeStruct((NC, NPAD, HID), jnp.float32),
        mesh=mesh,
        compiler_params=pltpu.CompilerParams(needs_layout_passes=False),
        scratch_types=_SCB_SCRATCH
        + [pltpu.VMEM_SHARED((NPAD, HID), jnp.float32)],
    )


def _scb2_body(pkE, wAll, z2, outP, *rest):
    bufs, acc_sh = rest[:-1], rest[-1]
    rowsA = bufs[8]
    c = lax.axis_index("c")
    s = lax.axis_index("s")
    wid = c * NS + s
    rbase = s * _RPT
    _zero_vmem_2d(rowsA, 128, HID)
    for b in range(_RPT // 128):
        pltpu.sync_copy(rowsA, acc_sh.at[pl.ds(rbase + b * 128, 128)])
    plsc.subcore_barrier()
    _phase_b_pipe(pkE.at[wid], wAll.at[wid], bufs, acc_sh, z2,
                  _CH2, HID, 0)
    plsc.subcore_barrier()
    _acc_readout(acc_sh, rowsA, lambda r: outP.at[c, pl.ds(r, 128)], rbase)


_SC_CACHE = {}


def _sc_kernels():
    if "k" not in _SC_CACHE:
        mesh = plsc.VectorSubcoreMesh(core_axis_name="c", subcore_axis_name="s",
                                      num_cores=NC, num_subcores=NS)
        _SC_CACHE["k"] = (
            _sca1_def(mesh)(_sca1_body),
            _scb1_def(mesh)(_scb1_body),
            _sca2_def(mesh)(_sca2_body),
            _scb2_def(mesh)(_scb2_body),
        )
    return _SC_CACHE["k"]


def kernel(h, edge_index, W1, a1, W2, a2):
    _sca1, _scb1, _sca2, _scb2 = _sc_kernels()
    h = h.astype(jnp.float32)
    src = edge_index[0].astype(jnp.int32)
    dst = edge_index[1].astype(jnp.int32)
    npad_e = EP - E
    srcp = jnp.concatenate([src, jnp.zeros((npad_e,), jnp.int32)])
    dstp = jnp.concatenate([dst, jnp.full((npad_e,), NPAD - 1, jnp.int32)])
    pk = srcp * (1 << PKSH) + dstp
    pk1 = pk.reshape(NS, _CH1, 128)
    pk2 = pk.reshape(NC * NS, _CH2, 128)

    hp = jnp.pad(h, ((0, NPAD - N), (0, 0)))
    U = jnp.einsum("hio,ho->ih", W1, a1[:, :HID])
    V = jnp.einsum("hio,ho->ih", W1, a1[:, HID:])
    UV = jnp.concatenate([U, V], axis=1)

    z1, S1 = _tc1(hp, W1, UV)
    ssT = S1[:, :HEADS].T
    sdT = S1[:, HEADS:].T
    z1flat = z1.reshape(HEADS * NPAD, HID)

    w1, den1p = _sca1(pk1, ssT, sdT)
    out_raw = _scb1(pk1, w1, z1flat)

    W2r = W2.reshape(HEADS, HID, OUT)
    A2p = jnp.pad(jnp.stack([a2[0, :OUT], a2[0, OUT:]], axis=1),
                  ((0, 0), (0, 6)))
    den1t = jnp.transpose(den1p, (0, 2, 1))
    z2, S2 = _tc2(out_raw, den1t, W2r, A2p)

    w2, den2p = _sca2(pk2, S2[:, 0], S2[:, 1])
    z2p = jnp.pad(z2, ((0, 0), (0, HID - OUT)))
    out2p = _scb2(pk2, w2, z2p)
    out = _tc3(out2p, den2p.T)
    return out[:N]
